# baseline (speedup 1.0000x reference)
import functools

import jax
import jax.numpy as jnp
from jax import lax
from jax.experimental import pallas as pl
from jax.experimental.pallas import tpu as pltpu
from jax.experimental.pallas import tpu_sc as plsc

VOCAB = 50
DIM = 16
BATCH = 4096
FIELDS = 26

_NC = 2
_NS = 16
_NW = _NC * _NS

_RPW = BATCH // _NW
_GPR = _RPW // DIM
_NITER = FIELDS * _GPR
_CT = DIM // 8
_ROWS = FIELDS * _CT * _NW * 8


@functools.partial(
    pl.kernel,
    out_type=jax.ShapeDtypeStruct((_ROWS, 128), jnp.float32),
    mesh=plsc.VectorSubcoreMesh(core_axis_name="c", subcore_axis_name="s"),
    scratch_types=[
        pltpu.VMEM((VOCAB * DIM,), jnp.float32),
        pltpu.VMEM((_RPW * FIELDS,), jnp.int32),
        pltpu.VMEM((FIELDS * DIM, _RPW), jnp.float32),
        pltpu.SemaphoreType.DMA,
        pltpu.SemaphoreType.DMA,
    ],
    compiler_params=pltpu.CompilerParams(
        use_tc_tiling_on_sc=False, needs_layout_passes=False),
)
def _gather_kernel(table_hbm, idx_hbm, out_hbm, tab_l, idx_l, out_v,
                   in_sem, out_sem):
    wid = lax.axis_index("s") * _NC + lax.axis_index("c")
    with jax.named_scope("stage_in"):
        c_tab = pltpu.async_copy(table_hbm, tab_l, in_sem)
        c_idx = pltpu.async_copy(
            idx_hbm.at[pl.ds(wid * _RPW * FIELDS, _RPW * FIELDS)], idx_l,
            in_sem)
        c_tab.wait()
        c_idx.wait()

    lanes26 = lax.iota(jnp.int32, DIM) * FIELDS

    with jax.named_scope("gather"):
        @plsc.parallel_loop(0, _NITER, unroll=8)
        def step(i):
            f = lax.shift_right_logical(i, 3)
            gr = lax.bitwise_and(i, 7)
            rb = gr * (DIM * FIELDS) + f
            v = plsc.load_gather(idx_l, [lanes26 + rb])
            a0 = v * DIM
            for c in range(DIM):
                w = plsc.load_gather(tab_l, [a0 + c])
                out_v[f * DIM + c, pl.ds(gr * DIM, DIM)] = w

    with jax.named_scope("stream_out"):
        copies = [
            pltpu.async_copy(out_v.at[pl.ds(k * 8, 8)],
                             out_hbm.at[pl.ds(k * (_NW * 8) + wid * 8, 8)],
                             out_sem)
            for k in range(FIELDS * _CT)
        ]
        for c in copies:
            c.wait()


def kernel(table, token_ids):
    raw = _gather_kernel(table.reshape(-1), token_ids.reshape(-1))
    out = raw.reshape(FIELDS, _CT, _NW, 8, _RPW)
    return out.transpose(2, 4, 0, 1, 3).reshape(BATCH, FIELDS, DIM)


# speedup vs baseline: 1.0625x; 1.0625x over previous
import functools

import jax
import jax.numpy as jnp
from jax import lax
from jax.experimental import pallas as pl
from jax.experimental.pallas import tpu as pltpu
from jax.experimental.pallas import tpu_sc as plsc

VOCAB = 50
DIM = 16
BATCH = 4096
FIELDS = 26

_NC = 2
_NS = 16
_NW = _NC * _NS

_RPW = BATCH // _NW
_GPR = _RPW // DIM
_NITER = FIELDS * _GPR
_CT = DIM // 8
_ROWS = FIELDS * _CT * _NW * 8


@functools.partial(
    pl.kernel,
    out_type=jax.ShapeDtypeStruct((_ROWS, 128), jnp.float32),
    mesh=plsc.VectorSubcoreMesh(core_axis_name="c", subcore_axis_name="s"),
    scratch_types=[
        pltpu.VMEM((VOCAB * DIM,), jnp.float32),
        pltpu.VMEM((_RPW * FIELDS,), jnp.int32),
        pltpu.VMEM((FIELDS * DIM, _RPW), jnp.float32),
        pltpu.SemaphoreType.DMA,
        pltpu.SemaphoreType.DMA,
    ],
    compiler_params=pltpu.CompilerParams(
        use_tc_tiling_on_sc=False, needs_layout_passes=False),
)
def _gather_kernel(table_hbm, idx_hbm, out_hbm, tab_l, idx_l, out_v,
                   in_sem, out_sem):
    wid = lax.axis_index("s") * _NC + lax.axis_index("c")
    with jax.named_scope("stage_in"):
        c_tab = pltpu.async_copy(table_hbm, tab_l, in_sem)
        c_idx = pltpu.async_copy(
            idx_hbm.at[pl.ds(wid * _RPW * FIELDS, _RPW * FIELDS)], idx_l,
            in_sem)
        c_tab.wait()
        c_idx.wait()

    lanes26 = lax.iota(jnp.int32, DIM) * FIELDS

    with jax.named_scope("gather"):
        @plsc.parallel_loop(0, _NITER, unroll=4)
        def step(i):
            f = lax.shift_right_logical(i, 3)
            gr = lax.bitwise_and(i, 7)
            rb = gr * (DIM * FIELDS) + f
            v = plsc.load_gather(idx_l, [lanes26 + rb])
            a0 = v * DIM
            for c in range(DIM):
                w = plsc.load_gather(tab_l, [a0 + c])
                out_v[f * DIM + c, pl.ds(gr * DIM, DIM)] = w

    with jax.named_scope("stream_out"):
        copies = [
            pltpu.async_copy(out_v.at[pl.ds(k * 8, 8)],
                             out_hbm.at[pl.ds(k * (_NW * 8) + wid * 8, 8)],
                             out_sem)
            for k in range(FIELDS * _CT)
        ]
        for c in copies:
            c.wait()


def kernel(table, token_ids):
    raw = _gather_kernel(table.reshape(-1), token_ids.reshape(-1))
    out = raw.reshape(FIELDS, _CT, _NW, 8, _RPW)
    return out.transpose(2, 4, 0, 1, 3).reshape(BATCH, FIELDS, DIM)


# speedup vs baseline: 1.3327x; 1.2543x over previous
import functools

import jax
import jax.numpy as jnp
from jax import lax
from jax.experimental import pallas as pl
from jax.experimental.pallas import tpu as pltpu
from jax.experimental.pallas import tpu_sc as plsc

VOCAB = 50
DIM = 16
BATCH = 4096
FIELDS = 26

_NC = 2
_NS = 16
_NW = _NC * _NS

_VPAD = 64
_RPW = BATCH // _NW
_GPR = _RPW // DIM
_NITER = FIELDS * _GPR
_CT = DIM // 8
_ROWS = FIELDS * _CT * _NW * 8


@functools.partial(
    pl.kernel,
    out_type=jax.ShapeDtypeStruct((_ROWS, 128), jnp.float32),
    mesh=plsc.VectorSubcoreMesh(core_axis_name="c", subcore_axis_name="s"),
    scratch_types=[
        pltpu.VMEM((_VPAD * DIM,), jnp.float32),
        pltpu.VMEM((DIM * _VPAD,), jnp.float32),
        pltpu.VMEM((_RPW * FIELDS,), jnp.int32),
        pltpu.VMEM((FIELDS * DIM, _RPW), jnp.float32),
        pltpu.SemaphoreType.DMA,
        pltpu.SemaphoreType.DMA,
    ],
    compiler_params=pltpu.CompilerParams(
        use_tc_tiling_on_sc=False, needs_layout_passes=False),
)
def _gather_kernel(table_hbm, idx_hbm, out_hbm, tab_l, tab_t, idx_l, out_v,
                   in_sem, out_sem):
    wid = lax.axis_index("s") * _NC + lax.axis_index("c")
    with jax.named_scope("stage_in"):
        c_tab = pltpu.async_copy(table_hbm, tab_l.at[pl.ds(0, VOCAB * DIM)],
                                 in_sem)
        c_idx = pltpu.async_copy(
            idx_hbm.at[pl.ds(wid * _RPW * FIELDS, _RPW * FIELDS)], idx_l,
            in_sem)
        c_tab.wait()
        c_idx.wait()

    with jax.named_scope("transpose"):
        for c in range(DIM):
            for g in range(_VPAD // DIM):
                src = (lax.iota(jnp.int32, DIM) + g * DIM) * DIM + c
                w = plsc.load_gather(tab_l, [src])
                tab_t[pl.ds(c * _VPAD + g * DIM, DIM)] = w

    lanes26 = lax.iota(jnp.int32, DIM) * FIELDS

    with jax.named_scope("gather"):
        @plsc.parallel_loop(0, _NITER, unroll=4)
        def step(i):
            f = lax.shift_right_logical(i, 3)
            gr = lax.bitwise_and(i, 7)
            rb = gr * (DIM * FIELDS) + f
            v = plsc.load_gather(idx_l, [lanes26 + rb])
            for c in range(DIM):
                w = plsc.load_gather(tab_t, [v + c * _VPAD])
                out_v[f * DIM + c, pl.ds(gr * DIM, DIM)] = w

    with jax.named_scope("stream_out"):
        copies = [
            pltpu.async_copy(out_v.at[pl.ds(k * 8, 8)],
                             out_hbm.at[pl.ds(k * (_NW * 8) + wid * 8, 8)],
                             out_sem)
            for k in range(FIELDS * _CT)
        ]
        for c in copies:
            c.wait()


def kernel(table, token_ids):
    raw = _gather_kernel(table.reshape(-1), token_ids.reshape(-1))
    out = raw.reshape(FIELDS, _CT, _NW, 8, _RPW)
    return out.transpose(2, 4, 0, 1, 3).reshape(BATCH, FIELDS, DIM)


# speedup vs baseline: 1.3373x; 1.0035x over previous
import functools

import jax
import jax.numpy as jnp
from jax import lax
from jax.experimental import pallas as pl
from jax.experimental.pallas import tpu as pltpu
from jax.experimental.pallas import tpu_sc as plsc

VOCAB = 50
DIM = 16
BATCH = 4096
FIELDS = 26

_NC = 2
_NS = 16
_NW = _NC * _NS

_VPAD = 64
_RPW = BATCH // _NW
_GPR = _RPW // DIM
_NITER = FIELDS * _GPR
_CT = DIM // 8
_ROWS = FIELDS * _CT * _NW * 8


@functools.partial(
    pl.kernel,
    out_type=jax.ShapeDtypeStruct((_ROWS, 128), jnp.float32),
    mesh=plsc.VectorSubcoreMesh(core_axis_name="c", subcore_axis_name="s"),
    scratch_types=[
        pltpu.VMEM((_VPAD * DIM,), jnp.float32),
        pltpu.VMEM((DIM * _VPAD,), jnp.float32),
        pltpu.VMEM((_RPW * FIELDS,), jnp.int32),
        pltpu.VMEM((FIELDS * DIM, _RPW), jnp.float32),
        pltpu.SemaphoreType.DMA,
        pltpu.SemaphoreType.DMA,
    ],
    compiler_params=pltpu.CompilerParams(
        use_tc_tiling_on_sc=False, needs_layout_passes=False),
)
def _gather_kernel(table_hbm, idx_hbm, out_hbm, tab_l, tab_t, idx_v, out_v,
                   in_sem, out_sem):
    wid = lax.axis_index("s") * _NC + lax.axis_index("c")
    with jax.named_scope("stage_in"):
        c_tab = pltpu.async_copy(table_hbm, tab_l.at[pl.ds(0, VOCAB * DIM)],
                                 in_sem)
        c_idx = pltpu.async_copy(
            idx_hbm.at[pl.ds(wid * _RPW * FIELDS, _RPW * FIELDS)], idx_v,
            in_sem)
        c_tab.wait()

    with jax.named_scope("transpose"):
        for c in range(DIM):
            for g in range(_VPAD // DIM):
                src = (lax.iota(jnp.int32, DIM) + g * DIM) * DIM + c
                w = plsc.load_gather(tab_l, [src])
                tab_t[pl.ds(c * _VPAD + g * DIM, DIM)] = w
        c_idx.wait()

    lanes26 = lax.iota(jnp.int32, DIM) * FIELDS

    with jax.named_scope("gather"):
        @plsc.parallel_loop(0, _NITER, unroll=4)
        def step(i):
            f = lax.shift_right_logical(i, 3)
            gr = lax.bitwise_and(i, 7)
            rb = gr * (DIM * FIELDS) + f
            v = plsc.load_gather(idx_v, [lanes26 + rb])
            for c in range(DIM):
                w = plsc.load_gather(tab_t, [v + c * _VPAD])
                out_v[f * DIM + c, pl.ds(gr * DIM, DIM)] = w

    with jax.named_scope("stream_out"):
        copies = [
            pltpu.async_copy(out_v.at[pl.ds(k * 8, 8)],
                             out_hbm.at[pl.ds(k * (_NW * 8) + wid * 8, 8)],
                             out_sem)
            for k in range(FIELDS * _CT)
        ]
        for c in copies:
            c.wait()


def kernel(table, token_ids):
    raw = _gather_kernel(table.reshape(-1), token_ids.reshape(-1))
    out = raw.reshape(FIELDS, _CT, _NW, 8, _RPW)
    return out.transpose(2, 4, 0, 1, 3).reshape(BATCH, FIELDS, DIM)
